# baseline (speedup 1.0000x reference)
import functools

import jax
import jax.numpy as jnp
from jax.experimental import pallas as pl
from jax.experimental.pallas import tpu as pltpu

B, S, D, DS = 4, 2048, 1024, 64
NF, NR, NT = 64, 32, 48
N_ALL = NF + NR + NT
N_OUT = NF + 2 * NR + NT
TKF, TKR, TKT = 8, 4, 6

SEGW = 128
N_PAD = 3 * SEGW
OF_F, OF_R, OF_T = 0, SEGW, 2 * SEGW

BLK = 1024
NS = S // BLK


def _topk_mask_normalize(w, k, n):
    wt = jnp.swapaxes(w, 0, 1)
    il = jax.lax.broadcasted_iota(jnp.int32, (1, n), 1)
    jt = jax.lax.broadcasted_iota(jnp.int32, (n, 1), 0)
    beats = (wt > w) | ((wt == w) & (jt < il))
    rank = jnp.sum(beats.astype(jnp.float32), axis=0, keepdims=True)
    sparse = jnp.where(rank < k, w, 0.0)
    return sparse / (jnp.sum(sparse, axis=1, keepdims=True) + 1e-8)


def _router_kernel(x_ref, imp_ref, w_ref, b_ref, emb_ref, embt_ref,
                   out_ref, p_buf, ent_s):
    b = pl.program_id(0)
    s = pl.program_id(1)

    @pl.when(s == 0)
    def _():
        em = emb_ref[...]
        nr = jnp.sqrt(jnp.sum(em * em, axis=1, keepdims=True))
        nrt = jnp.swapaxes(nr, 0, 1) + 1e-12
        entn = embt_ref[...] / nrt
        ent_s[...] = jnp.zeros_like(ent_s)
        ent_s[:, OF_F:OF_F + NF] = entn[:, :NF]
        ent_s[:, OF_R:OF_R + NR] = entn[:, NF:NF + NR]
        ent_s[:, OF_T:OF_T + NT] = entn[:, NF + NR:]

    h = jnp.dot(x_ref[0], w_ref[...],
                preferred_element_type=jnp.float32)
    h = h + b_ref[...]
    al = jnp.dot(h, ent_s[...],
                 preferred_element_type=jnp.float32)

    lane = jax.lax.broadcasted_iota(jnp.int32, (BLK, N_PAD), 1)

    def bc3(vf, vr, vt):
        return jnp.where(lane < SEGW, vf,
                         jnp.where(lane < 2 * SEGW, vr, vt))

    m_bc = bc3(jnp.max(al[:, OF_F:OF_F + NF], axis=1, keepdims=True),
               jnp.max(al[:, OF_R:OF_R + NR], axis=1, keepdims=True),
               jnp.max(al[:, OF_T:OF_T + NT], axis=1, keepdims=True))
    e = jnp.exp(al - m_bc)
    s_bc = bc3(jnp.sum(e[:, OF_F:OF_F + NF], axis=1, keepdims=True),
               jnp.sum(e[:, OF_R:OF_R + NR], axis=1, keepdims=True),
               jnp.sum(e[:, OF_T:OF_T + NT], axis=1, keepdims=True))
    p_buf[pl.ds(s * BLK, BLK), :] = e / s_bc

    @pl.when(s == NS - 1)
    def _():
        imp = imp_ref[pl.ds(b, 1), :]
        pooled = jnp.dot(imp, p_buf[...],
                         preferred_element_type=jnp.float32)
        out_ref[0] = pooled


@functools.partial(jax.jit, static_argnames=("interpret",))
def kernel(x, importance, W_proj, b_proj, neuron_emb, interpret=False):
    b2 = b_proj.reshape(1, DS)
    embt = neuron_emb.T

    out = pl.pallas_call(
        _router_kernel,
        grid=(B, NS),
        in_specs=[
            pl.BlockSpec((1, BLK, D), lambda b, s: (b, s, 0)),
            pl.BlockSpec((B, S), lambda b, s: (0, 0)),
            pl.BlockSpec((D, DS), lambda b, s: (0, 0)),
            pl.BlockSpec((1, DS), lambda b, s: (0, 0)),
            pl.BlockSpec((N_ALL, DS), lambda b, s: (0, 0)),
            pl.BlockSpec((DS, N_ALL), lambda b, s: (0, 0)),
        ],
        out_specs=pl.BlockSpec((1, 1, N_PAD), lambda b, s: (b, 0, 0)),
        out_shape=jax.ShapeDtypeStruct((B, 1, N_PAD), jnp.float32),
        scratch_shapes=[
            pltpu.VMEM((S, N_PAD), jnp.float32),
            pltpu.VMEM((DS, N_PAD), jnp.float32),
        ],
        compiler_params=pltpu.CompilerParams(
            dimension_semantics=("parallel", "arbitrary"),
        ),
        interpret=interpret,
    )(x, importance, W_proj, b2, neuron_emb, embt)

    return _sc_topk(out).reshape(B, N_OUT)


def _sc_task(pooled_hbm, out_hbm, wbuf, obuf, sem, b_i, seg_off, n, k, dsts):
    pltpu.async_copy(pooled_hbm.at[b_i, 0, pl.ds(seg_off, n)],
                     wbuf.at[pl.ds(0, n)], sem).wait()
    nc = n // 16
    idx = [jax.lax.iota(jnp.int32, 16) + 16 * c for c in range(nc)]
    orig = [wbuf[pl.ds(16 * c, 16)] for c in range(nc)]
    w = list(orig)
    sel = [jnp.zeros((16,), jnp.float32) for _ in range(nc)]
    BIG = jnp.int32(10000)
    NEG = jnp.float32(-3.0e38)
    for _ in range(k):
        m = w[0]
        for c in range(1, nc):
            m = jnp.maximum(m, w[c])
        mx = jnp.max(m)
        cand = [jnp.where(w[c] == mx, idx[c], BIG) for c in range(nc)]
        cm = cand[0]
        for c in range(1, nc):
            cm = jnp.minimum(cm, cand[c])
        j = jnp.min(cm)
        for c in range(nc):
            hit = idx[c] == j
            sel[c] = jnp.where(hit, 1.0, sel[c])
            w[c] = jnp.where(hit, NEG, w[c])
    sp = [orig[c] * sel[c] for c in range(nc)]
    acc = sp[0]
    for c in range(1, nc):
        acc = acc + sp[c]
    ssum = jnp.sum(acc) + 1e-8
    for c in range(nc):
        obuf[pl.ds(16 * c, 16)] = sp[c] / ssum
    for dst in dsts:
        pltpu.async_copy(obuf.at[pl.ds(0, n)],
                         out_hbm.at[b_i, 0, pl.ds(dst, n)], sem).wait()


def _sc_topk(pooled):
    import dataclasses
    from jax.experimental.pallas import tpu_sc as plsc

    mesh = plsc.VectorSubcoreMesh(core_axis_name="c", subcore_axis_name="s")
    cp = pltpu.CompilerParams()
    if "needs_layout_passes" in pltpu.CompilerParams.__dataclass_fields__:
        cp = dataclasses.replace(cp, needs_layout_passes=False)

    @functools.partial(
        pl.kernel,
        out_type=jax.ShapeDtypeStruct((B, 1, N_OUT), jnp.float32),
        mesh=mesh,
        compiler_params=cp,
        scratch_types=[pltpu.VMEM((64,), jnp.float32),
                       pltpu.VMEM((64,), jnp.float32),
                       pltpu.SemaphoreType.DMA],
    )
    def sc_kernel(pooled_hbm, out_hbm, wbuf, obuf, sem):
        cid = jax.lax.axis_index("c")
        sid = jax.lax.axis_index("s")
        tid = cid * 16 + sid
        segs = [
            (OF_F, NF, TKF, (0,)),
            (OF_R, NR, TKR, (NF, NF + NR)),
            (OF_T, NT, TKT, (NF + 2 * NR,)),
        ]
        for b_i in range(B):
            for gi, (seg_off, n, k, dsts) in enumerate(segs):
                @pl.when(tid == b_i * 3 + gi)
                def _(b_i=b_i, seg_off=seg_off, n=n, k=k, dsts=dsts):
                    _sc_task(pooled_hbm, out_hbm, wbuf, obuf, sem,
                             b_i, seg_off, n, k, dsts)

    return sc_kernel(pooled)


# speedup vs baseline: 1.6951x; 1.6951x over previous
import jax
import jax.numpy as jnp
from jax.experimental import pallas as pl
from jax.experimental.pallas import tpu as pltpu

B, S, D, DS = 4, 2048, 1024, 64
NF, NR, NT = 64, 32, 48
N_ALL = NF + NR + NT
N_OUT = NF + 2 * NR + NT
TKF, TKR, TKT = 8, 4, 6

SEGW = 128
N_PAD = 3 * SEGW
OF_F, OF_R, OF_T = 0, SEGW, 2 * SEGW

BLK = 1024
NS = S // BLK


def _topk_mask_normalize(w, k, n):
    wt = jnp.swapaxes(w, 0, 1)
    il = jax.lax.broadcasted_iota(jnp.int32, (1, n), 1)
    jt = jax.lax.broadcasted_iota(jnp.int32, (n, 1), 0)
    beats = (wt > w) | ((wt == w) & (jt < il))
    rank = jnp.sum(beats.astype(jnp.float32), axis=0, keepdims=True)
    sparse = jnp.where(rank < k, w, 0.0)
    return sparse / (jnp.sum(sparse, axis=1, keepdims=True) + 1e-8)


def _router_kernel(x_ref, imp_ref, w_ref, b_ref, emb_ref, embt_ref,
                   out_ref, p_buf, ent_s):
    b = pl.program_id(0)
    s = pl.program_id(1)

    @pl.when(s == 0)
    def _():
        em = emb_ref[...]
        nr = jnp.sqrt(jnp.sum(em * em, axis=1, keepdims=True))
        nrt = jnp.swapaxes(nr, 0, 1) + 1e-12
        entn = embt_ref[...] / nrt
        ent_s[...] = jnp.zeros_like(ent_s)
        ent_s[:, OF_F:OF_F + NF] = entn[:, :NF]
        ent_s[:, OF_R:OF_R + NR] = entn[:, NF:NF + NR]
        ent_s[:, OF_T:OF_T + NT] = entn[:, NF + NR:]

    h = jnp.dot(x_ref[0], w_ref[...],
                preferred_element_type=jnp.float32)
    h = h + b_ref[...]
    al = jnp.dot(h, ent_s[...],
                 preferred_element_type=jnp.float32)

    lane = jax.lax.broadcasted_iota(jnp.int32, (BLK, N_PAD), 1)

    def bc3(vf, vr, vt):
        return jnp.where(lane < SEGW, vf,
                         jnp.where(lane < 2 * SEGW, vr, vt))

    m_bc = bc3(jnp.max(al[:, OF_F:OF_F + NF], axis=1, keepdims=True),
               jnp.max(al[:, OF_R:OF_R + NR], axis=1, keepdims=True),
               jnp.max(al[:, OF_T:OF_T + NT], axis=1, keepdims=True))
    e = jnp.exp(al - m_bc)
    s_bc = bc3(jnp.sum(e[:, OF_F:OF_F + NF], axis=1, keepdims=True),
               jnp.sum(e[:, OF_R:OF_R + NR], axis=1, keepdims=True),
               jnp.sum(e[:, OF_T:OF_T + NT], axis=1, keepdims=True))
    p_buf[pl.ds(s * BLK, BLK), :] = e / s_bc

    @pl.when(s == NS - 1)
    def _():
        imp = imp_ref[pl.ds(b, 1), :]
        pooled = jnp.dot(imp, p_buf[...],
                         preferred_element_type=jnp.float32)
        wf = _topk_mask_normalize(pooled[:, OF_F:OF_F + NF], TKF, NF)
        wr = _topk_mask_normalize(pooled[:, OF_R:OF_R + NR], TKR, NR)
        wt = _topk_mask_normalize(pooled[:, OF_T:OF_T + NT], TKT, NT)
        out_ref[0, :, 0:NF] = wf
        out_ref[0, :, NF:NF + NR] = wr
        out_ref[0, :, NF + NR:NF + 2 * NR] = wr
        out_ref[0, :, NF + 2 * NR:] = wt


@jax.jit
def kernel(x, importance, W_proj, b_proj, neuron_emb):
    b2 = b_proj.reshape(1, DS)
    embt = neuron_emb.T

    out = pl.pallas_call(
        _router_kernel,
        grid=(B, NS),
        in_specs=[
            pl.BlockSpec((1, BLK, D), lambda b, s: (b, s, 0)),
            pl.BlockSpec((B, S), lambda b, s: (0, 0)),
            pl.BlockSpec((D, DS), lambda b, s: (0, 0)),
            pl.BlockSpec((1, DS), lambda b, s: (0, 0)),
            pl.BlockSpec((N_ALL, DS), lambda b, s: (0, 0)),
            pl.BlockSpec((DS, N_ALL), lambda b, s: (0, 0)),
        ],
        out_specs=pl.BlockSpec((1, 1, N_OUT), lambda b, s: (b, 0, 0)),
        out_shape=jax.ShapeDtypeStruct((B, 1, N_OUT), jnp.float32),
        scratch_shapes=[
            pltpu.VMEM((S, N_PAD), jnp.float32),
            pltpu.VMEM((DS, N_PAD), jnp.float32),
        ],
        compiler_params=pltpu.CompilerParams(
            dimension_semantics=("parallel", "arbitrary"),
        ),
    )(x, importance, W_proj, b2, neuron_emb, embt)

    return out.reshape(B, N_OUT)
